# baseline (speedup 1.0000x reference)
import functools

import jax
import jax.numpy as jnp
from jax import lax
from jax.experimental import pallas as pl
from jax.experimental.pallas import tpu as pltpu
from jax.experimental.pallas import tpu_sc as plsc


def _rdot(a, b):
    return lax.dot_general(a.astype(jnp.bfloat16), b.astype(jnp.bfloat16),
                           (((1,), (0,)), ((), ())),
                           preferred_element_type=jnp.float32)


def _bdot(a, b):
    return lax.dot_general(a.astype(jnp.bfloat16), b.astype(jnp.bfloat16),
                           (((0,), (0,)), ((), ())),
                           preferred_element_type=jnp.float32)


def _fe_body(nblocks, pts_ref, w_in_ref, b_in_ref, w_blk_ref, b_blk_ref,
             feats_ref, gmax_ref, psq_ref):
    j = pl.program_id(1)
    x = pts_ref[0]
    psq_ref[0] = jnp.sum(x * x, axis=1, keepdims=True)
    f = jnp.maximum(_rdot(x, w_in_ref[...]) + b_in_ref[...], 0.0)
    fs = [f]
    for i in range(nblocks):
        f = jnp.maximum(_rdot(f, w_blk_ref[i]) + b_blk_ref[i], 0.0)
        fs.append(f)
    feats_ref[0] = jnp.concatenate(fs, axis=1)
    m = jnp.max(f, axis=0, keepdims=True)

    @pl.when(j == 0)
    def _():
        gmax_ref[0] = m

    @pl.when(j != 0)
    def _():
        gmax_ref[0] = jnp.maximum(gmax_ref[0], m)


def _knn_body(n, k, pts_ref, q_ref, psq_ref, gi_ref, ws_ref):
    b = pl.program_id(0)
    p = pts_ref[0]
    q = q_ref[0]
    psq = psq_ref[0]
    qsq = jnp.sum(q * q, axis=0, keepdims=True)
    pq = _bdot(p, q)
    d2 = (qsq - 2.0 * pq) + psq
    iota = lax.broadcasted_iota(jnp.int32, d2.shape, 0)

    rows = []
    vals = []
    for _ in range(k):
        v = jnp.min(d2, axis=0, keepdims=True)
        m = d2 == v
        i = jnp.min(jnp.where(m, iota, n), axis=0, keepdims=True)
        d2 = jnp.where(m, jnp.inf, d2)
        rows.append(i)
        vals.append(v)

    ws = [1.0 / (jnp.maximum(v, 0.0) + 1e-8) for v in vals]
    wsum = ws[0] + ws[1] + ws[2]
    ws = [w / wsum for w in ws]
    gi = jnp.concatenate(rows, axis=0) + b * n
    gi_ref[...] = gi[:, None, None, :]
    ws_ref[...] = jnp.concatenate(ws, axis=0)[:, None, None, :]


def _sc_interp_body(qpw, ch, d, table_ref, gi_ref, ws_ref, out_ref,
                    ia0, ia1, ia2, ib0, ib1, ib2,
                    wa0, wa1, wa2, wb0, wb1, wb2,
                    ra0, ra1, ra2, rb0, rb1, rb2,
                    ob, sema, semb):
    wid = lax.axis_index("s") * 2 + lax.axis_index("c")
    base = wid * qpw
    nch = qpw // ch

    def fetch(ci, iv, wv, rr, sem):
        qb = base + ci * ch
        for kk in range(3):
            pltpu.sync_copy(gi_ref.at[kk, pl.ds(qb, ch)], iv[kk])
            pltpu.sync_copy(ws_ref.at[kk, pl.ds(qb, ch)], wv[kk])
        for kk in range(3):
            pltpu.async_copy(table_ref.at[iv[kk]], rr[kk], sem)

    def drain(iv, rr, sem):
        for kk in range(3):
            pltpu.make_async_copy(table_ref.at[iv[kk]], rr[kk], sem).wait()

    def compute(ci, wv, rr):
        qb = base + ci * ch
        r0, r1, r2 = rr

        def per_query(qi, carry2):
            a0 = wv[0][qi, :]
            a1 = wv[1][qi, :]
            a2 = wv[2][qi, :]
            for v in range(d // 16):
                sl = pl.ds(v * 16, 16)
                acc = (r0[qi, sl] * a0 + r1[qi, sl] * a1
                       + r2[qi, sl] * a2)
                ob[qi, sl] = acc
            return carry2

        lax.fori_loop(0, ch, per_query, 0)
        pltpu.sync_copy(ob, out_ref.at[pl.ds(qb, ch)])

    iva, ivb = [ia0, ia1, ia2], [ib0, ib1, ib2]
    wva, wvb = [wa0, wa1, wa2], [wb0, wb1, wb2]
    rra, rrb = [ra0, ra1, ra2], [rb0, rb1, rb2]

    fetch(0, iva, wva, rra, sema)

    def outer(g, carry):
        c0 = 2 * g
        c1 = 2 * g + 1
        fetch(c1, ivb, wvb, rrb, semb)
        drain(iva, rra, sema)
        compute(c0, wva, rra)

        @pl.when(g < nch // 2 - 1)
        def _():
            fetch(c0 + 2, iva, wva, rra, sema)

        drain(ivb, rrb, semb)
        compute(c1, wvb, rrb)
        return carry

    lax.fori_loop(0, nch // 2, outer, 0)


def _reg_body(interp_ref, q_ref, gmax_ref, wr1q_ref, wr1l_ref, wr1g_ref,
              wr2_ref, wr3_ref, br1_ref, br2_ref, br3_ref, out_ref):
    interp = interp_ref[...]
    h1 = (_rdot(interp, wr1l_ref[...])
          + _rdot(q_ref[0], wr1q_ref[...])
          + _rdot(gmax_ref[0], wr1g_ref[...])
          + br1_ref[...])
    h1 = jnp.maximum(h1, 0.0)
    h2 = jnp.maximum(_rdot(h1, wr2_ref[...]) + br2_ref[...], 0.0)
    h2b = h2.astype(jnp.bfloat16).astype(jnp.float32)
    w3 = wr3_ref[...].astype(jnp.bfloat16).astype(jnp.float32)
    out_ref[...] = (jnp.sum(h2b * w3, axis=1, keepdims=True)
                    + br3_ref[...])


def _full(shape):
    nd = len(shape)
    return pl.BlockSpec(shape, lambda *_: (0,) * nd)


@jax.jit
def kernel(original_pts, query_pts, W_in, b_in, W_blk, b_blk,
           Wr1, br1, Wr2, br2, Wr3, br3):
    B, _, N = original_pts.shape
    M = query_pts.shape[2]
    C = W_in.shape[1]
    BN = W_blk.shape[0]
    H = Wr1.shape[1]
    K = 3
    C4 = (BN + 1) * C

    NB = 1024
    MB = 512
    MB3 = 512
    NW = 32
    CH = 16
    QPW = (B * M) // NW

    pts_t = original_pts.transpose(0, 2, 1)

    feats, gmax, psq = pl.pallas_call(
        functools.partial(_fe_body, BN),
        grid=(B, N // NB),
        in_specs=[
            pl.BlockSpec((1, NB, 3), lambda b, j: (b, j, 0)),
            _full((3, C)),
            _full((1, C)),
            _full((BN, C, C)),
            _full((BN, 1, C)),
        ],
        out_specs=[
            pl.BlockSpec((1, NB, C4), lambda b, j: (b, j, 0)),
            pl.BlockSpec((1, 1, C), lambda b, j: (b, 0, 0)),
            pl.BlockSpec((1, NB, 1), lambda b, j: (b, j, 0)),
        ],
        out_shape=[
            jax.ShapeDtypeStruct((B, N, C4), jnp.float32),
            jax.ShapeDtypeStruct((B, 1, C), jnp.float32),
            jax.ShapeDtypeStruct((B, N, 1), jnp.float32),
        ],
    )(pts_t, W_in, b_in.reshape(1, C), W_blk, b_blk.reshape(BN, 1, C))

    gi4, ws4 = pl.pallas_call(
        functools.partial(_knn_body, N, K),
        grid=(B, M // MB),
        in_specs=[
            pl.BlockSpec((1, 3, N), lambda b, j: (b, 0, 0)),
            pl.BlockSpec((1, 3, MB), lambda b, j: (b, 0, j)),
            pl.BlockSpec((1, N, 1), lambda b, j: (b, 0, 0)),
        ],
        out_specs=[
            pl.BlockSpec((K, 1, 1, MB), lambda b, j: (0, b, 0, j)),
            pl.BlockSpec((K, 1, 1, MB), lambda b, j: (0, b, 0, j)),
        ],
        out_shape=[
            jax.ShapeDtypeStruct((K, B, 1, M), jnp.int32),
            jax.ShapeDtypeStruct((K, B, 1, M), jnp.float32),
        ],
    )(original_pts, query_pts, psq)

    feats_flat = feats.reshape(B * N, C4)
    gi_flat = gi4.reshape(K, B * M)
    ws_exp = jnp.broadcast_to(ws4.reshape(K, B * M)[:, :, None],
                              (K, B * M, 16))

    mesh = plsc.VectorSubcoreMesh(core_axis_name="c", subcore_axis_name="s")
    sc_interp = functools.partial(
        pl.kernel,
        mesh=mesh,
        out_type=jax.ShapeDtypeStruct((B * M, C4), jnp.float32),
        scratch_types=(
            [pltpu.VMEM((CH,), jnp.int32)] * 6
            + [pltpu.VMEM((CH, 16), jnp.float32)] * 6
            + [pltpu.VMEM((CH, C4), jnp.float32)] * 7
            + [pltpu.SemaphoreType.DMA, pltpu.SemaphoreType.DMA]
        ),
    )(functools.partial(_sc_interp_body, QPW, CH, C4))
    interp = sc_interp(feats_flat, gi_flat, ws_exp)

    q_t = query_pts.transpose(0, 2, 1)
    Wr1q = Wr1[:3]
    Wr1l = Wr1[3:3 + C4]
    Wr1g = Wr1[3 + C4:]
    jb = M // MB3

    out = pl.pallas_call(
        _reg_body,
        grid=(B, jb),
        in_specs=[
            pl.BlockSpec((MB3, C4), lambda b, j: (b * jb + j, 0)),
            pl.BlockSpec((1, MB3, 3), lambda b, j: (b, j, 0)),
            pl.BlockSpec((1, 1, C), lambda b, j: (b, 0, 0)),
            _full((3, H)),
            _full((C4, H)),
            _full((C, H)),
            _full((H, H)),
            _full((1, H)),
            _full((1, H)),
            _full((1, H)),
            _full((1, 1)),
        ],
        out_specs=pl.BlockSpec((MB3, 1), lambda b, j: (b * jb + j, 0)),
        out_shape=jax.ShapeDtypeStruct((B * M, 1), jnp.float32),
    )(interp, q_t, gmax, Wr1q, Wr1l, Wr1g, Wr2,
      Wr3.reshape(1, H), br1.reshape(1, H), br2.reshape(1, H),
      br3.reshape(1, 1))

    return out.reshape(B, 1, M)


# speedup vs baseline: 1.1714x; 1.1714x over previous
import functools

import jax
import jax.numpy as jnp
from jax import lax
from jax.experimental import pallas as pl
from jax.experimental.pallas import tpu as pltpu
from jax.experimental.pallas import tpu_sc as plsc


def _rdot(a, b):
    return lax.dot_general(a.astype(jnp.bfloat16), b.astype(jnp.bfloat16),
                           (((1,), (0,)), ((), ())),
                           preferred_element_type=jnp.float32)


def _bdot(a, b):
    return lax.dot_general(a.astype(jnp.bfloat16), b.astype(jnp.bfloat16),
                           (((0,), (0,)), ((), ())),
                           preferred_element_type=jnp.float32)


def _fe_body(nblocks, pts_ref, w_in_ref, b_in_ref, w_blk_ref, b_blk_ref,
             feats_ref, gmax_ref, psq_ref):
    j = pl.program_id(1)
    x = pts_ref[0]
    psq_ref[0] = jnp.sum(x * x, axis=1, keepdims=True)
    f = jnp.maximum(_rdot(x, w_in_ref[...]) + b_in_ref[...], 0.0)
    fs = [f]
    for i in range(nblocks):
        f = jnp.maximum(_rdot(f, w_blk_ref[i]) + b_blk_ref[i], 0.0)
        fs.append(f)
    feats_ref[0] = jnp.concatenate(fs, axis=1)
    m = jnp.max(f, axis=0, keepdims=True)

    @pl.when(j == 0)
    def _():
        gmax_ref[0] = m

    @pl.when(j != 0)
    def _():
        gmax_ref[0] = jnp.maximum(gmax_ref[0], m)


def _knn_body(n, k, pts_ref, q_ref, psq_ref, gi_ref, ws_ref):
    b = pl.program_id(0)
    p = pts_ref[0]
    q = q_ref[0]
    psq = psq_ref[0]
    qsq = jnp.sum(q * q, axis=0, keepdims=True)
    pq = _bdot(p, q)
    d2 = (qsq - 2.0 * pq) + psq
    iota = lax.broadcasted_iota(jnp.int32, d2.shape, 0)

    rows = []
    vals = []
    for _ in range(k):
        v = jnp.min(d2, axis=0, keepdims=True)
        m = d2 == v
        i = jnp.min(jnp.where(m, iota, n), axis=0, keepdims=True)
        d2 = jnp.where(m, jnp.inf, d2)
        rows.append(i)
        vals.append(v)

    ws = [1.0 / (jnp.maximum(v, 0.0) + 1e-8) for v in vals]
    wsum = ws[0] + ws[1] + ws[2]
    ws = [w / wsum for w in ws]
    gi = jnp.concatenate(rows, axis=0) + b * n
    gi_ref[...] = gi[:, None, None, :]
    ws_ref[...] = jnp.concatenate(ws, axis=0)[:, None, None, :]


def _sc_interp_body(qpw, ch, d, table_ref, gi_ref, ws_ref, out_ref,
                    i0, i1, i2, w0, w1, w2,
                    ra0, ra1, ra2, rb0, rb1, rb2,
                    ob, sema, semb):
    wid = lax.axis_index("s") * 2 + lax.axis_index("c")
    nq = qpw * 32
    half = qpw // 2
    nch = half // ch
    ii = [i0, i1, i2]
    ww = [w0, w1, w2]
    rra, rrb = [ra0, ra1, ra2], [rb0, rb1, rb2]

    for h in range(2):
        hb = wid * qpw + h * half
        for kk in range(3):
            pltpu.sync_copy(gi_ref.at[pl.ds(kk * nq + hb, half)], ii[kk])
            pltpu.sync_copy(ws_ref.at[pl.ds(kk * nq + hb, half)], ww[kk])

        def fetch(ci, rr, sem):
            qo = ci * ch
            for kk in range(3):
                pltpu.async_copy(table_ref.at[ii[kk].at[pl.ds(qo, ch)]],
                                 rr[kk], sem)

        def drain(ci, rr, sem):
            qo = ci * ch
            for kk in range(3):
                pltpu.make_async_copy(
                    table_ref.at[ii[kk].at[pl.ds(qo, ch)]],
                    rr[kk], sem).wait()

        def compute(ci, rr, hb=hb):
            qo = ci * ch
            r0, r1, r2 = rr

            def per_query(qi, carry2):
                a0 = w0[qo + qi, :]
                a1 = w1[qo + qi, :]
                a2 = w2[qo + qi, :]
                for v in range(d // 16):
                    sl = pl.ds(v * 16, 16)
                    acc = (r0[qi, sl] * a0 + r1[qi, sl] * a1
                           + r2[qi, sl] * a2)
                    ob[qi, sl] = acc
                return carry2

            lax.fori_loop(0, ch, per_query, 0)
            pltpu.sync_copy(ob, out_ref.at[pl.ds(hb + qo, ch)])

        fetch(0, rra, sema)

        def outer(g, carry, fetch=fetch, drain=drain, compute=compute):
            c0 = 2 * g
            c1 = 2 * g + 1
            fetch(c1, rrb, semb)
            drain(c0, rra, sema)
            compute(c0, rra)

            @pl.when(g < nch // 2 - 1)
            def _():
                fetch(c0 + 2, rra, sema)

            drain(c1, rrb, semb)
            compute(c1, rrb)
            return carry

        lax.fori_loop(0, nch // 2, outer, 0)


def _reg_body(interp_ref, q_ref, gmax_ref, wr1q_ref, wr1l_ref, wr1g_ref,
              wr2_ref, wr3_ref, br1_ref, br2_ref, br3_ref, out_ref):
    interp = interp_ref[...]
    h1 = (_rdot(interp, wr1l_ref[...])
          + _rdot(q_ref[0], wr1q_ref[...])
          + _rdot(gmax_ref[0], wr1g_ref[...])
          + br1_ref[...])
    h1 = jnp.maximum(h1, 0.0)
    h2 = jnp.maximum(_rdot(h1, wr2_ref[...]) + br2_ref[...], 0.0)
    h2b = h2.astype(jnp.bfloat16).astype(jnp.float32)
    w3 = wr3_ref[...].astype(jnp.bfloat16).astype(jnp.float32)
    out_ref[...] = (jnp.sum(h2b * w3, axis=1, keepdims=True)
                    + br3_ref[...])


def _full(shape):
    nd = len(shape)
    return pl.BlockSpec(shape, lambda *_: (0,) * nd)


@jax.jit
def kernel(original_pts, query_pts, W_in, b_in, W_blk, b_blk,
           Wr1, br1, Wr2, br2, Wr3, br3):
    B, _, N = original_pts.shape
    M = query_pts.shape[2]
    C = W_in.shape[1]
    BN = W_blk.shape[0]
    H = Wr1.shape[1]
    K = 3
    C4 = (BN + 1) * C

    NB = 1024
    MB = 512
    MB3 = 512
    NW = 32
    CH = 8
    QPW = (B * M) // NW

    pts_t = original_pts.transpose(0, 2, 1)

    feats, gmax, psq = pl.pallas_call(
        functools.partial(_fe_body, BN),
        grid=(B, N // NB),
        in_specs=[
            pl.BlockSpec((1, NB, 3), lambda b, j: (b, j, 0)),
            _full((3, C)),
            _full((1, C)),
            _full((BN, C, C)),
            _full((BN, 1, C)),
        ],
        out_specs=[
            pl.BlockSpec((1, NB, C4), lambda b, j: (b, j, 0)),
            pl.BlockSpec((1, 1, C), lambda b, j: (b, 0, 0)),
            pl.BlockSpec((1, NB, 1), lambda b, j: (b, j, 0)),
        ],
        out_shape=[
            jax.ShapeDtypeStruct((B, N, C4), jnp.float32),
            jax.ShapeDtypeStruct((B, 1, C), jnp.float32),
            jax.ShapeDtypeStruct((B, N, 1), jnp.float32),
        ],
    )(pts_t, W_in, b_in.reshape(1, C), W_blk, b_blk.reshape(BN, 1, C))

    gi4, ws4 = pl.pallas_call(
        functools.partial(_knn_body, N, K),
        grid=(B, M // MB),
        in_specs=[
            pl.BlockSpec((1, 3, N), lambda b, j: (b, 0, 0)),
            pl.BlockSpec((1, 3, MB), lambda b, j: (b, 0, j)),
            pl.BlockSpec((1, N, 1), lambda b, j: (b, 0, 0)),
        ],
        out_specs=[
            pl.BlockSpec((K, 1, 1, MB), lambda b, j: (0, b, 0, j)),
            pl.BlockSpec((K, 1, 1, MB), lambda b, j: (0, b, 0, j)),
        ],
        out_shape=[
            jax.ShapeDtypeStruct((K, B, 1, M), jnp.int32),
            jax.ShapeDtypeStruct((K, B, 1, M), jnp.float32),
        ],
    )(original_pts, query_pts, psq)

    feats_flat = feats.reshape(B * N, C4)
    gi_flat = gi4.reshape(K * B * M)
    ws_exp = jnp.broadcast_to(ws4.reshape(K * B * M)[:, None],
                              (K * B * M, 16))

    mesh = plsc.VectorSubcoreMesh(core_axis_name="c", subcore_axis_name="s")
    sc_interp = functools.partial(
        pl.kernel,
        mesh=mesh,
        out_type=jax.ShapeDtypeStruct((B * M, C4), jnp.float32),
        scratch_types=(
            [pltpu.VMEM((QPW // 2,), jnp.int32)] * 3
            + [pltpu.VMEM((QPW // 2, 16), jnp.float32)] * 3
            + [pltpu.VMEM((CH, C4), jnp.float32)] * 7
            + [pltpu.SemaphoreType.DMA, pltpu.SemaphoreType.DMA]
        ),
    )(functools.partial(_sc_interp_body, QPW, CH, C4))
    interp = sc_interp(feats_flat, gi_flat, ws_exp)

    q_t = query_pts.transpose(0, 2, 1)
    Wr1q = Wr1[:3]
    Wr1l = Wr1[3:3 + C4]
    Wr1g = Wr1[3 + C4:]
    jb = M // MB3

    out = pl.pallas_call(
        _reg_body,
        grid=(B, jb),
        in_specs=[
            pl.BlockSpec((MB3, C4), lambda b, j: (b * jb + j, 0)),
            pl.BlockSpec((1, MB3, 3), lambda b, j: (b, j, 0)),
            pl.BlockSpec((1, 1, C), lambda b, j: (b, 0, 0)),
            _full((3, H)),
            _full((C4, H)),
            _full((C, H)),
            _full((H, H)),
            _full((1, H)),
            _full((1, H)),
            _full((1, H)),
            _full((1, 1)),
        ],
        out_specs=pl.BlockSpec((MB3, 1), lambda b, j: (b * jb + j, 0)),
        out_shape=jax.ShapeDtypeStruct((B * M, 1), jnp.float32),
    )(interp, q_t, gmax, Wr1q, Wr1l, Wr1g, Wr2,
      Wr3.reshape(1, H), br1.reshape(1, H), br2.reshape(1, H),
      br3.reshape(1, 1))

    return out.reshape(B, 1, M)


# speedup vs baseline: 1.2205x; 1.0420x over previous
import functools

import jax
import jax.numpy as jnp
from jax import lax
from jax.experimental import pallas as pl
from jax.experimental.pallas import tpu as pltpu
from jax.experimental.pallas import tpu_sc as plsc


def _rdot(a, b):
    return lax.dot_general(a.astype(jnp.bfloat16), b.astype(jnp.bfloat16),
                           (((1,), (0,)), ((), ())),
                           preferred_element_type=jnp.float32)


def _bdot(a, b):
    return lax.dot_general(a.astype(jnp.bfloat16), b.astype(jnp.bfloat16),
                           (((0,), (0,)), ((), ())),
                           preferred_element_type=jnp.float32)


def _fe_body(nblocks, pts_ref, w_in_ref, b_in_ref, w_blk_ref, b_blk_ref,
             feats_ref, gmax_ref, psq_ref):
    j = pl.program_id(1)
    x = pts_ref[0]
    psq_ref[0] = jnp.sum(x * x, axis=1, keepdims=True)
    f = jnp.maximum(_rdot(x, w_in_ref[...]) + b_in_ref[...], 0.0)
    fs = [f]
    for i in range(nblocks):
        f = jnp.maximum(_rdot(f, w_blk_ref[i]) + b_blk_ref[i], 0.0)
        fs.append(f)
    feats_ref[0] = jnp.concatenate(fs, axis=1)
    m = jnp.max(f, axis=0, keepdims=True)

    @pl.when(j == 0)
    def _():
        gmax_ref[0] = m

    @pl.when(j != 0)
    def _():
        gmax_ref[0] = jnp.maximum(gmax_ref[0], m)


def _knn_body(n, k, pts_ref, q_ref, psq_ref, gi_ref, ws_ref):
    b = pl.program_id(0)
    p = pts_ref[0]
    q = q_ref[0]
    psq = psq_ref[0]
    qsq = jnp.sum(q * q, axis=0, keepdims=True)
    pq = _bdot(p, q)
    d2 = (qsq - 2.0 * pq) + psq
    iota = lax.broadcasted_iota(jnp.int32, d2.shape, 0)

    rows = []
    vals = []
    for _ in range(k):
        v = jnp.min(d2, axis=0, keepdims=True)
        m = d2 == v
        i = jnp.min(jnp.where(m, iota, n), axis=0, keepdims=True)
        d2 = jnp.where(m, jnp.inf, d2)
        rows.append(i)
        vals.append(v)

    ws = [1.0 / (jnp.maximum(v, 0.0) + 1e-8) for v in vals]
    wsum = ws[0] + ws[1] + ws[2]
    ws = [w / wsum for w in ws]
    gi = jnp.concatenate(rows, axis=0) + b * n
    gi_ref[...] = gi[:, None, None, :]
    ws_ref[...] = jnp.concatenate(ws, axis=0)[:, None, None, :]


def _sc_interp_body(qpw, ch, d, table_ref, gi_ref, ws_ref, out_ref,
                    i0, i1, i2, w0, w1, w2,
                    ra0, ra1, ra2, rb0, rb1, rb2,
                    ob, sema, semb):
    wid = lax.axis_index("s") * 2 + lax.axis_index("c")
    nq = qpw * 32
    half = qpw // 2
    nch = half // ch
    ii = [i0, i1, i2]
    ww = [w0, w1, w2]
    rra, rrb = [ra0, ra1, ra2], [rb0, rb1, rb2]

    for h in range(2):
        hb = wid * qpw + h * half
        for kk in range(3):
            pltpu.sync_copy(gi_ref.at[pl.ds(kk * nq + hb, half)], ii[kk])
            pltpu.sync_copy(ws_ref.at[pl.ds(kk * nq + hb, half)], ww[kk])

        def fetch(ci, rr, sem):
            qo = ci * ch
            for kk in range(3):
                pltpu.async_copy(table_ref.at[ii[kk].at[pl.ds(qo, ch)]],
                                 rr[kk], sem)

        def drain(ci, rr, sem):
            qo = ci * ch
            for kk in range(3):
                pltpu.make_async_copy(
                    table_ref.at[ii[kk].at[pl.ds(qo, ch)]],
                    rr[kk], sem).wait()

        def compute(ci, rr, hb=hb):
            qo = ci * ch
            r0, r1, r2 = rr

            def per_query(qi, carry2):
                a0 = w0[qo + qi, :]
                a1 = w1[qo + qi, :]
                a2 = w2[qo + qi, :]
                for v in range(d // 16):
                    sl = pl.ds(v * 16, 16)
                    acc = (r0[qi, sl] * a0 + r1[qi, sl] * a1
                           + r2[qi, sl] * a2)
                    ob[qi, sl] = acc
                return carry2

            lax.fori_loop(0, ch, per_query, 0)
            pltpu.sync_copy(ob, out_ref.at[pl.ds(hb + qo, ch)])

        fetch(0, rra, sema)

        def outer(g, carry, fetch=fetch, drain=drain, compute=compute):
            c0 = 2 * g
            c1 = 2 * g + 1
            fetch(c1, rrb, semb)
            drain(c0, rra, sema)
            compute(c0, rra)

            @pl.when(g < nch // 2 - 1)
            def _():
                fetch(c0 + 2, rra, sema)

            drain(c1, rrb, semb)
            compute(c1, rrb)
            return carry

        lax.fori_loop(0, nch // 2, outer, 0)


def _reg_body(interp_ref, q_ref, gmax_ref, wr1q_ref, wr1l_ref, wr1g_ref,
              wr2_ref, wr3_ref, br1_ref, br2_ref, br3_ref, out_ref):
    interp = interp_ref[...]
    h1 = (_rdot(interp, wr1l_ref[...])
          + _rdot(q_ref[0], wr1q_ref[...])
          + _rdot(gmax_ref[0], wr1g_ref[...])
          + br1_ref[...])
    h1 = jnp.maximum(h1, 0.0)
    h2 = jnp.maximum(_rdot(h1, wr2_ref[...]) + br2_ref[...], 0.0)
    h2b = h2.astype(jnp.bfloat16).astype(jnp.float32)
    w3 = wr3_ref[...].astype(jnp.bfloat16).astype(jnp.float32)
    out_ref[...] = (jnp.sum(h2b * w3, axis=1, keepdims=True)
                    + br3_ref[...])


def _full(shape):
    nd = len(shape)
    return pl.BlockSpec(shape, lambda *_: (0,) * nd)


@jax.jit
def kernel(original_pts, query_pts, W_in, b_in, W_blk, b_blk,
           Wr1, br1, Wr2, br2, Wr3, br3):
    B, _, N = original_pts.shape
    M = query_pts.shape[2]
    C = W_in.shape[1]
    BN = W_blk.shape[0]
    H = Wr1.shape[1]
    K = 3
    C4 = (BN + 1) * C

    NB = 1024
    MB = 1024
    MB3 = 512
    NW = 32
    CH = 8
    QPW = (B * M) // NW

    pts_t = original_pts.transpose(0, 2, 1)

    feats, gmax, psq = pl.pallas_call(
        functools.partial(_fe_body, BN),
        grid=(B, N // NB),
        in_specs=[
            pl.BlockSpec((1, NB, 3), lambda b, j: (b, j, 0)),
            _full((3, C)),
            _full((1, C)),
            _full((BN, C, C)),
            _full((BN, 1, C)),
        ],
        out_specs=[
            pl.BlockSpec((1, NB, C4), lambda b, j: (b, j, 0)),
            pl.BlockSpec((1, 1, C), lambda b, j: (b, 0, 0)),
            pl.BlockSpec((1, NB, 1), lambda b, j: (b, j, 0)),
        ],
        out_shape=[
            jax.ShapeDtypeStruct((B, N, C4), jnp.float32),
            jax.ShapeDtypeStruct((B, 1, C), jnp.float32),
            jax.ShapeDtypeStruct((B, N, 1), jnp.float32),
        ],
    )(pts_t, W_in, b_in.reshape(1, C), W_blk, b_blk.reshape(BN, 1, C))

    gi4, ws4 = pl.pallas_call(
        functools.partial(_knn_body, N, K),
        grid=(B, M // MB),
        in_specs=[
            pl.BlockSpec((1, 3, N), lambda b, j: (b, 0, 0)),
            pl.BlockSpec((1, 3, MB), lambda b, j: (b, 0, j)),
            pl.BlockSpec((1, N, 1), lambda b, j: (b, 0, 0)),
        ],
        out_specs=[
            pl.BlockSpec((K, 1, 1, MB), lambda b, j: (0, b, 0, j)),
            pl.BlockSpec((K, 1, 1, MB), lambda b, j: (0, b, 0, j)),
        ],
        out_shape=[
            jax.ShapeDtypeStruct((K, B, 1, M), jnp.int32),
            jax.ShapeDtypeStruct((K, B, 1, M), jnp.float32),
        ],
    )(original_pts, query_pts, psq)

    feats_flat = feats.reshape(B * N, C4)
    gi_flat = gi4.reshape(K * B * M)
    ws_exp = jnp.broadcast_to(ws4.reshape(K * B * M)[:, None],
                              (K * B * M, 16))

    mesh = plsc.VectorSubcoreMesh(core_axis_name="c", subcore_axis_name="s")
    sc_interp = functools.partial(
        pl.kernel,
        mesh=mesh,
        out_type=jax.ShapeDtypeStruct((B * M, C4), jnp.float32),
        scratch_types=(
            [pltpu.VMEM((QPW // 2,), jnp.int32)] * 3
            + [pltpu.VMEM((QPW // 2, 16), jnp.float32)] * 3
            + [pltpu.VMEM((CH, C4), jnp.float32)] * 7
            + [pltpu.SemaphoreType.DMA, pltpu.SemaphoreType.DMA]
        ),
    )(functools.partial(_sc_interp_body, QPW, CH, C4))
    interp = sc_interp(feats_flat, gi_flat, ws_exp)

    q_t = query_pts.transpose(0, 2, 1)
    Wr1q = Wr1[:3]
    Wr1l = Wr1[3:3 + C4]
    Wr1g = Wr1[3 + C4:]
    jb = M // MB3

    out = pl.pallas_call(
        _reg_body,
        grid=(B, jb),
        in_specs=[
            pl.BlockSpec((MB3, C4), lambda b, j: (b * jb + j, 0)),
            pl.BlockSpec((1, MB3, 3), lambda b, j: (b, j, 0)),
            pl.BlockSpec((1, 1, C), lambda b, j: (b, 0, 0)),
            _full((3, H)),
            _full((C4, H)),
            _full((C, H)),
            _full((H, H)),
            _full((1, H)),
            _full((1, H)),
            _full((1, H)),
            _full((1, 1)),
        ],
        out_specs=pl.BlockSpec((MB3, 1), lambda b, j: (b * jb + j, 0)),
        out_shape=jax.ShapeDtypeStruct((B * M, 1), jnp.float32),
    )(interp, q_t, gmax, Wr1q, Wr1l, Wr1g, Wr2,
      Wr3.reshape(1, H), br1.reshape(1, H), br2.reshape(1, H),
      br3.reshape(1, 1))

    return out.reshape(B, 1, M)


# speedup vs baseline: 1.2451x; 1.0201x over previous
import functools

import jax
import jax.numpy as jnp
from jax import lax
from jax.experimental import pallas as pl
from jax.experimental.pallas import tpu as pltpu
from jax.experimental.pallas import tpu_sc as plsc


def _rdot(a, b):
    return lax.dot_general(a.astype(jnp.bfloat16), b.astype(jnp.bfloat16),
                           (((1,), (0,)), ((), ())),
                           preferred_element_type=jnp.float32)


def _bdot(a, b):
    return lax.dot_general(a.astype(jnp.bfloat16), b.astype(jnp.bfloat16),
                           (((0,), (0,)), ((), ())),
                           preferred_element_type=jnp.float32)


def _fe_body(nblocks, pts_ref, w_in_ref, b_in_ref, w_blk_ref, b_blk_ref,
             feats_ref, gmax_ref, psq_ref):
    j = pl.program_id(1)
    x = pts_ref[0]
    psq_ref[0] = jnp.sum(x * x, axis=1, keepdims=True)
    f = jnp.maximum(_rdot(x, w_in_ref[...]) + b_in_ref[...], 0.0)
    fs = [f]
    for i in range(nblocks):
        f = jnp.maximum(_rdot(f, w_blk_ref[i]) + b_blk_ref[i], 0.0)
        fs.append(f)
    feats_ref[0] = jnp.concatenate(fs, axis=1)
    m = jnp.max(f, axis=0, keepdims=True)

    @pl.when(j == 0)
    def _():
        gmax_ref[0] = m

    @pl.when(j != 0)
    def _():
        gmax_ref[0] = jnp.maximum(gmax_ref[0], m)


def _knn_body(n, k, pts_ref, q_ref, psq_ref, gi_ref, ws_ref):
    b = pl.program_id(0)
    p = pts_ref[0]
    q = q_ref[0]
    psq = psq_ref[0]
    qsq = jnp.sum(q * q, axis=0, keepdims=True)
    pq = _bdot(p, q)
    d2 = (qsq - 2.0 * pq) + psq
    iota = lax.broadcasted_iota(jnp.int32, d2.shape, 0)

    rows = []
    vals = []
    for _ in range(k):
        v = jnp.min(d2, axis=0, keepdims=True)
        m = d2 == v
        i = jnp.min(jnp.where(m, iota, n), axis=0, keepdims=True)
        d2 = jnp.where(m, jnp.inf, d2)
        rows.append(i)
        vals.append(v)

    ws = [1.0 / (jnp.maximum(v, 0.0) + 1e-8) for v in vals]
    wsum = ws[0] + ws[1] + ws[2]
    ws = [w / wsum for w in ws]
    gi = jnp.concatenate(rows, axis=0) + b * n
    gi_ref[...] = gi[:, None, None, :]
    ws_ref[...] = jnp.concatenate(ws, axis=0)[:, None, None, :]


def _sc_interp_body(qpw, ch, d, table_ref, gi_ref, ws_ref, out_ref,
                    i0, i1, i2, w0, w1, w2,
                    ra0, ra1, ra2, rb0, rb1, rb2,
                    ob, sema, semb):
    wid = lax.axis_index("s") * 2 + lax.axis_index("c")
    nq = qpw * 32
    half = qpw // 2
    nch = half // ch
    ii = [i0, i1, i2]
    ww = [w0, w1, w2]
    rra, rrb = [ra0, ra1, ra2], [rb0, rb1, rb2]

    for h in range(2):
        hb = wid * qpw + h * half
        for kk in range(3):
            pltpu.sync_copy(gi_ref.at[pl.ds(kk * nq + hb, half)], ii[kk])
            pltpu.sync_copy(ws_ref.at[pl.ds(kk * nq + hb, half)], ww[kk])

        def fetch(ci, rr, sem):
            qo = ci * ch
            for kk in range(3):
                pltpu.async_copy(table_ref.at[ii[kk].at[pl.ds(qo, ch)]],
                                 rr[kk], sem)

        def drain(ci, rr, sem):
            qo = ci * ch
            for kk in range(3):
                pltpu.make_async_copy(
                    table_ref.at[ii[kk].at[pl.ds(qo, ch)]],
                    rr[kk], sem).wait()

        def compute(ci, rr, hb=hb):
            qo = ci * ch
            r0, r1, r2 = rr

            def per_query(qi, carry2):
                a0 = w0[qo + qi, :]
                a1 = w1[qo + qi, :]
                a2 = w2[qo + qi, :]
                for v in range(d // 16):
                    sl = pl.ds(v * 16, 16)
                    acc = (r0[qi, sl] * a0 + r1[qi, sl] * a1
                           + r2[qi, sl] * a2)
                    ob[qi, sl] = acc
                return carry2

            lax.fori_loop(0, ch, per_query, 0)
            pltpu.sync_copy(ob, out_ref.at[pl.ds(hb + qo, ch)])

        fetch(0, rra, sema)

        def outer(g, carry, fetch=fetch, drain=drain, compute=compute):
            c0 = 2 * g
            c1 = 2 * g + 1
            fetch(c1, rrb, semb)
            drain(c0, rra, sema)
            compute(c0, rra)

            @pl.when(g < nch // 2 - 1)
            def _():
                fetch(c0 + 2, rra, sema)

            drain(c1, rrb, semb)
            compute(c1, rrb)
            return carry

        lax.fori_loop(0, nch // 2, outer, 0)


def _reg_body(interp_ref, q_ref, gmax_ref, wr1q_ref, wr1l_ref, wr1g_ref,
              wr2_ref, wr3_ref, br1_ref, br2_ref, br3_ref, out_ref):
    interp = interp_ref[...]
    h1 = (_rdot(interp, wr1l_ref[...])
          + _rdot(q_ref[0], wr1q_ref[...])
          + _rdot(gmax_ref[0], wr1g_ref[...])
          + br1_ref[...])
    h1 = jnp.maximum(h1, 0.0)
    h2 = jnp.maximum(_rdot(h1, wr2_ref[...]) + br2_ref[...], 0.0)
    h2b = h2.astype(jnp.bfloat16).astype(jnp.float32)
    w3 = wr3_ref[...].astype(jnp.bfloat16).astype(jnp.float32)
    out_ref[...] = (jnp.sum(h2b * w3, axis=1, keepdims=True)
                    + br3_ref[...])


def _full(shape):
    nd = len(shape)
    return pl.BlockSpec(shape, lambda *_: (0,) * nd)


@jax.jit
def kernel(original_pts, query_pts, W_in, b_in, W_blk, b_blk,
           Wr1, br1, Wr2, br2, Wr3, br3):
    B, _, N = original_pts.shape
    M = query_pts.shape[2]
    C = W_in.shape[1]
    BN = W_blk.shape[0]
    H = Wr1.shape[1]
    K = 3
    C4 = (BN + 1) * C

    NB = 1024
    MB = 2048
    MB3 = 512
    NW = 32
    CH = 8
    QPW = (B * M) // NW

    pts_t = original_pts.transpose(0, 2, 1)

    feats, gmax, psq = pl.pallas_call(
        functools.partial(_fe_body, BN),
        grid=(B, N // NB),
        in_specs=[
            pl.BlockSpec((1, NB, 3), lambda b, j: (b, j, 0)),
            _full((3, C)),
            _full((1, C)),
            _full((BN, C, C)),
            _full((BN, 1, C)),
        ],
        out_specs=[
            pl.BlockSpec((1, NB, C4), lambda b, j: (b, j, 0)),
            pl.BlockSpec((1, 1, C), lambda b, j: (b, 0, 0)),
            pl.BlockSpec((1, NB, 1), lambda b, j: (b, j, 0)),
        ],
        out_shape=[
            jax.ShapeDtypeStruct((B, N, C4), jnp.float32),
            jax.ShapeDtypeStruct((B, 1, C), jnp.float32),
            jax.ShapeDtypeStruct((B, N, 1), jnp.float32),
        ],
    )(pts_t, W_in, b_in.reshape(1, C), W_blk, b_blk.reshape(BN, 1, C))

    gi4, ws4 = pl.pallas_call(
        functools.partial(_knn_body, N, K),
        grid=(B, M // MB),
        in_specs=[
            pl.BlockSpec((1, 3, N), lambda b, j: (b, 0, 0)),
            pl.BlockSpec((1, 3, MB), lambda b, j: (b, 0, j)),
            pl.BlockSpec((1, N, 1), lambda b, j: (b, 0, 0)),
        ],
        out_specs=[
            pl.BlockSpec((K, 1, 1, MB), lambda b, j: (0, b, 0, j)),
            pl.BlockSpec((K, 1, 1, MB), lambda b, j: (0, b, 0, j)),
        ],
        out_shape=[
            jax.ShapeDtypeStruct((K, B, 1, M), jnp.int32),
            jax.ShapeDtypeStruct((K, B, 1, M), jnp.float32),
        ],
    )(original_pts, query_pts, psq)

    feats_flat = feats.reshape(B * N, C4)
    gi_flat = gi4.reshape(K * B * M)
    ws_exp = jnp.broadcast_to(ws4.reshape(K * B * M)[:, None],
                              (K * B * M, 16))

    mesh = plsc.VectorSubcoreMesh(core_axis_name="c", subcore_axis_name="s")
    sc_interp = functools.partial(
        pl.kernel,
        mesh=mesh,
        out_type=jax.ShapeDtypeStruct((B * M, C4), jnp.float32),
        scratch_types=(
            [pltpu.VMEM((QPW // 2,), jnp.int32)] * 3
            + [pltpu.VMEM((QPW // 2, 16), jnp.float32)] * 3
            + [pltpu.VMEM((CH, C4), jnp.float32)] * 7
            + [pltpu.SemaphoreType.DMA, pltpu.SemaphoreType.DMA]
        ),
    )(functools.partial(_sc_interp_body, QPW, CH, C4))
    interp = sc_interp(feats_flat, gi_flat, ws_exp)

    q_t = query_pts.transpose(0, 2, 1)
    Wr1q = Wr1[:3]
    Wr1l = Wr1[3:3 + C4]
    Wr1g = Wr1[3 + C4:]
    jb = M // MB3

    out = pl.pallas_call(
        _reg_body,
        grid=(B, jb),
        in_specs=[
            pl.BlockSpec((MB3, C4), lambda b, j: (b * jb + j, 0)),
            pl.BlockSpec((1, MB3, 3), lambda b, j: (b, j, 0)),
            pl.BlockSpec((1, 1, C), lambda b, j: (b, 0, 0)),
            _full((3, H)),
            _full((C4, H)),
            _full((C, H)),
            _full((H, H)),
            _full((1, H)),
            _full((1, H)),
            _full((1, H)),
            _full((1, 1)),
        ],
        out_specs=pl.BlockSpec((MB3, 1), lambda b, j: (b * jb + j, 0)),
        out_shape=jax.ShapeDtypeStruct((B * M, 1), jnp.float32),
    )(interp, q_t, gmax, Wr1q, Wr1l, Wr1g, Wr2,
      Wr3.reshape(1, H), br1.reshape(1, H), br2.reshape(1, H),
      br3.reshape(1, 1))

    return out.reshape(B, 1, M)
